# baseline (speedup 1.0000x reference)
import functools

import jax
import jax.numpy as jnp
import numpy as np
from jax import lax
from jax.experimental import pallas as pl
from jax.experimental.pallas import tpu as pltpu
from jax.experimental.pallas import tpu_sc as plsc

_CARDS = [100000] * 26
_F = len(_CARDS)
_D = 32
_B = 16384
_NFLAT = _B * _F

_NC, _NS, _L = 2, 16, 16
_NW = _NC * _NS
_PER_W = _NFLAT // _NW

_CHUNK_ROWS = 64
_CH = _CHUNK_ROWS * _F
_NCH = _PER_W // _CH
_NG = _CH // 128
_PAT = 8 * _F


def _body(x_hbm, off_hbm, bias_hbm, emb_hbm, out_hbm,
          x_v, idx_v, rows_v, off_v, bias_v, sem):
    cid = lax.axis_index("c")
    sid = lax.axis_index("s")
    wid = sid * _NC + cid
    base = wid * _PER_W

    pltpu.sync_copy(off_hbm, off_v)
    pltpu.sync_copy(bias_hbm, bias_v)

    def chunk(c, carry):
        start = base + c * _CH
        pltpu.sync_copy(x_hbm.at[pl.ds(start, _CH)], x_v)
        for g in range(_CH // _PAT):
            for k in range(_PAT // _L):
                o = g * _PAT + k * _L
                vals = x_v[pl.ds(o, _L)] + off_v[pl.ds(k * _L, _L)]
                j = g * (_PAT // _L) + k
                idx_v[j // 8, pl.ds((j % 8) * _L, _L)] = vals.astype(jnp.int32)
        cops = [
            pltpu.async_copy(emb_hbm.at[idx_v.at[k]],
                             rows_v.at[pl.ds(k * 128, 128)], sem)
            for k in range(_NG)
        ]
        for cop in cops:
            cop.wait()
        def group(g, carry2):
            row0 = g * _F
            for r in range(_F):
                for h in range(2):
                    plsc.addupdate(rows_v.at[row0 + r, pl.ds(h * _L, _L)],
                                   bias_v[pl.ds(r * _D + h * _L, _L)])
            return carry2
        lax.fori_loop(0, _CHUNK_ROWS, group, 0)
        pltpu.sync_copy(rows_v, out_hbm.at[pl.ds(start, _CH)])
        return carry

    lax.fori_loop(0, _NCH, chunk, 0)


@jax.jit
def kernel(x, emb, bias):
    offsets = np.concatenate([[0], np.cumsum(_CARDS[:-1])]).astype(np.float32)
    off_pat = jnp.asarray(np.tile(offsets, _PAT // _F))
    x_flat = x.reshape(_NFLAT)
    bias_flat = bias.reshape(_F * _D)

    mesh = plsc.VectorSubcoreMesh(core_axis_name="c", subcore_axis_name="s")
    run = functools.partial(
        pl.kernel,
        out_type=jax.ShapeDtypeStruct((_NFLAT, _D), jnp.float32),
        mesh=mesh,
        compiler_params=pltpu.CompilerParams(use_tc_tiling_on_sc=False),
        scratch_types=[
            pltpu.VMEM((_CH,), jnp.float32),
            pltpu.VMEM((_NG, 128), jnp.int32),
            pltpu.VMEM((_CH, _D), jnp.float32),
            pltpu.VMEM((_PAT,), jnp.float32),
            pltpu.VMEM((_F * _D,), jnp.float32),
            pltpu.SemaphoreType.DMA,
        ],
    )(_body)
    out = run(x_flat, off_pat, bias_flat, emb)
    return out.reshape(_B, _F, _D)
